# baseline (speedup 1.0000x reference)
import jax
import jax.numpy as jnp
from jax import lax
from jax.experimental import pallas as pl
from jax.experimental.pallas import tpu as pltpu
from jax.experimental.pallas import tpu_sc as plsc

N_NODES = 10000
N_EDGES = 320000
L = 16
NC = 2
NS = 16
NPAD = 10240
SLICE = NPAD // NS
E_HIST = N_EDGES // NS
E_OUT = N_EDGES // (NC * NS)
TILE1 = 128
E_HIST_BUF = 20096
E_OUT_BUF = 10112


def _sc_body(pos_hbm, tail_hbm, out_hbm, hist_v, idx_v, tails_v, staged_v,
             red_v, outbuf_v, rows_sh, deg_sh, sem_tails, sem_pos, sem_out,
             sem_stage):
    cid = lax.axis_index("c")
    sid = lax.axis_index("s")
    wid = sid * NC + cid
    gbase = wid * E_OUT

    ones = jnp.ones((L,), jnp.float32)
    zeros = jnp.zeros((L,), jnp.float32)

    start_t = jnp.minimum((gbase // TILE1) * TILE1, N_EDGES - E_OUT_BUF)
    start_t = pl.multiple_of(start_t, TILE1)
    off_t = gbase - start_t

    start_h = jnp.minimum((sid * E_HIST // TILE1) * TILE1, N_EDGES - E_HIST_BUF)
    start_h = pl.multiple_of(start_h, TILE1)
    off_h = sid * E_HIST - start_h
    HSEGS = (80, 320, 320, 320, 210)
    HBOUND = (0, 1408, 6528, 11648, 16768, E_HIST_BUF)
    pos_cps = [
        pltpu.async_copy(
            pos_hbm.at[:, pl.ds(start_h + HBOUND[c], HBOUND[c + 1] - HBOUND[c])],
            idx_v.at[:, pl.ds(HBOUND[c], HBOUND[c + 1] - HBOUND[c])],
            sem_pos)
        for c in range(len(HSEGS))
    ]

    @plsc.parallel_loop(0, NPAD // L, unroll=4)
    def _zero(i):
        hist_v[pl.ds(i * L, L)] = zeros

    tails_cp = None
    base = 0
    for c in range(len(HSEGS)):
        pos_cps[c].wait()
        seg_base = base

        @plsc.parallel_loop(0, HSEGS[c], unroll=5)
        def _hist(e, _b=seg_base):
            plsc.addupdate_scatter(
                hist_v, [idx_v[1, pl.ds(off_h + (_b + e) * L, L)]], ones)
        base += HSEGS[c]
        if c == 1:
            tails_cp = pltpu.async_copy(
                tail_hbm.at[:, pl.ds(start_t, E_OUT_BUF)], tails_v, sem_tails)

    pltpu.sync_copy(hist_v, rows_sh.at[sid])
    plsc.subcore_barrier()

    col = sid * SLICE
    pltpu.sync_copy(rows_sh.at[:, pl.ds(col, SLICE)], staged_v)

    @plsc.parallel_loop(0, SLICE // L, unroll=2)
    def _reduce(j):
        acc = staged_v[0, pl.ds(j * L, L)]
        for r in range(1, NS):
            acc = acc + staged_v[r, pl.ds(j * L, L)]
        red_v[pl.ds(j * L, L)] = acc

    pltpu.sync_copy(red_v, deg_sh.at[pl.ds(col, SLICE)])
    plsc.subcore_barrier()

    pltpu.sync_copy(deg_sh, hist_v)
    tails_cp.wait()
    GSEGS = (160, 160, 160, 145)
    out_cps = []
    gb = 0
    for g in GSEGS:
        seg_base = gb

        @plsc.parallel_loop(0, g, unroll=5 if g == 145 else 4)
        def _gather(e, _b=seg_base):
            outbuf_v[pl.ds((_b + e) * L, L)] = plsc.load_gather(
                hist_v, [tails_v[1, pl.ds(off_t + (_b + e) * L, L)]])

        out_cps.append(pltpu.async_copy(
            outbuf_v.at[pl.ds(seg_base * L, g * L)],
            out_hbm.at[pl.ds(gbase + seg_base * L, g * L)], sem_out))
        gb += g
    for cp in out_cps:
        cp.wait()


@jax.jit
def _degree_gather(pos, tails):
    mesh = plsc.VectorSubcoreMesh(core_axis_name="c", subcore_axis_name="s")
    return pl.kernel(
        _sc_body,
        mesh=mesh,
        compiler_params=pltpu.CompilerParams(needs_layout_passes=False),
        out_type=jax.ShapeDtypeStruct((N_EDGES,), jnp.float32),
        scratch_types=[
            pltpu.VMEM((NPAD,), jnp.float32),
            pltpu.VMEM((2, E_HIST_BUF), jnp.int32),
            pltpu.VMEM((2, E_OUT_BUF), jnp.int32),
            pltpu.VMEM((NS, SLICE), jnp.float32),
            pltpu.VMEM((SLICE,), jnp.float32),
            pltpu.VMEM((E_OUT,), jnp.float32),
            pltpu.VMEM_SHARED((NS, NPAD), jnp.float32),
            pltpu.VMEM_SHARED((NPAD,), jnp.float32),
            pltpu.SemaphoreType.DMA,
            pltpu.SemaphoreType.DMA,
            pltpu.SemaphoreType.DMA,
            pltpu.SemaphoreType.DMA,
        ],
    )(pos, tails)


def kernel(z, edge_index, pos_edge_index):
    del z
    return _degree_gather(pos_edge_index.astype(jnp.int32),
                          edge_index.astype(jnp.int32))


# speedup vs baseline: 1.0090x; 1.0090x over previous
import jax
import jax.numpy as jnp
from jax import lax
from jax.experimental import pallas as pl
from jax.experimental.pallas import tpu as pltpu
from jax.experimental.pallas import tpu_sc as plsc

N_NODES = 10000
N_EDGES = 320000
L = 16
NC = 2
NS = 16
NPAD = 10240
SLICE = NPAD // NS
E_HIST = N_EDGES // NS
E_OUT = N_EDGES // (NC * NS)
TILE1 = 128
E_HIST_BUF = 20096
E_OUT_BUF = 10112


def _sc_body(pos_hbm, tail_hbm, out_hbm, hist_v, idx_v, tails_v, staged_v,
             red_v, outbuf_v, rows_sh, deg_sh, sem_tails, sem_pos, sem_out,
             sem_stage):
    cid = lax.axis_index("c")
    sid = lax.axis_index("s")
    wid = sid * NC + cid
    gbase = wid * E_OUT

    ones = jnp.ones((L,), jnp.float32)
    zeros = jnp.zeros((L,), jnp.float32)

    start_t = jnp.minimum((gbase // TILE1) * TILE1, N_EDGES - E_OUT_BUF)
    start_t = pl.multiple_of(start_t, TILE1)
    off_t = gbase - start_t

    start_h = jnp.minimum((sid * E_HIST // TILE1) * TILE1, N_EDGES - E_HIST_BUF)
    start_h = pl.multiple_of(start_h, TILE1)
    off_h = sid * E_HIST - start_h
    HSEGS = (80, 320, 320, 320, 210)
    HBOUND = (0, 1408, 6528, 11648, 16768, E_HIST_BUF)
    pos_cps = [
        pltpu.async_copy(
            pos_hbm.at[:, pl.ds(start_h + HBOUND[c], HBOUND[c + 1] - HBOUND[c])],
            idx_v.at[:, pl.ds(HBOUND[c], HBOUND[c + 1] - HBOUND[c])],
            sem_pos)
        for c in range(len(HSEGS))
    ]

    @plsc.parallel_loop(0, NPAD // L, unroll=4)
    def _zero(i):
        hist_v[pl.ds(i * L, L)] = zeros

    tails_cp = None
    base = 0
    for c in range(len(HSEGS)):
        pos_cps[c].wait()
        seg_base = base

        @plsc.parallel_loop(0, HSEGS[c], unroll=5)
        def _hist(e, _b=seg_base):
            plsc.addupdate_scatter(
                hist_v, [idx_v[1, pl.ds(off_h + (_b + e) * L, L)]], ones)
        base += HSEGS[c]
        if c == 1:
            tails_cp = pltpu.async_copy(
                tail_hbm.at[:, pl.ds(start_t, E_OUT_BUF)], tails_v, sem_tails)

    pltpu.sync_copy(hist_v, rows_sh.at[sid])
    plsc.subcore_barrier()

    col = sid * SLICE
    pltpu.sync_copy(rows_sh.at[:, pl.ds(col, SLICE)], staged_v)

    @plsc.parallel_loop(0, SLICE // L, unroll=2)
    def _reduce(j):
        acc = staged_v[0, pl.ds(j * L, L)]
        for r in range(1, NS):
            acc = acc + staged_v[r, pl.ds(j * L, L)]
        red_v[pl.ds(j * L, L)] = acc

    pltpu.sync_copy(red_v, deg_sh.at[pl.ds(col, SLICE)])
    plsc.subcore_barrier()

    pltpu.sync_copy(deg_sh, hist_v)
    tails_cp.wait()
    GSEGS = (320, 305)
    out_cps = []
    gb = 0
    for g in GSEGS:
        seg_base = gb

        @plsc.parallel_loop(0, g, unroll=5 if g == 305 else 4)
        def _gather(e, _b=seg_base):
            outbuf_v[pl.ds((_b + e) * L, L)] = plsc.load_gather(
                hist_v, [tails_v[1, pl.ds(off_t + (_b + e) * L, L)]])

        out_cps.append(pltpu.async_copy(
            outbuf_v.at[pl.ds(seg_base * L, g * L)],
            out_hbm.at[pl.ds(gbase + seg_base * L, g * L)], sem_out))
        gb += g
    for cp in out_cps:
        cp.wait()


@jax.jit
def _degree_gather(pos, tails):
    mesh = plsc.VectorSubcoreMesh(core_axis_name="c", subcore_axis_name="s")
    return pl.kernel(
        _sc_body,
        mesh=mesh,
        compiler_params=pltpu.CompilerParams(needs_layout_passes=False),
        out_type=jax.ShapeDtypeStruct((N_EDGES,), jnp.float32),
        scratch_types=[
            pltpu.VMEM((NPAD,), jnp.float32),
            pltpu.VMEM((2, E_HIST_BUF), jnp.int32),
            pltpu.VMEM((2, E_OUT_BUF), jnp.int32),
            pltpu.VMEM((NS, SLICE), jnp.float32),
            pltpu.VMEM((SLICE,), jnp.float32),
            pltpu.VMEM((E_OUT,), jnp.float32),
            pltpu.VMEM_SHARED((NS, NPAD), jnp.float32),
            pltpu.VMEM_SHARED((NPAD,), jnp.float32),
            pltpu.SemaphoreType.DMA,
            pltpu.SemaphoreType.DMA,
            pltpu.SemaphoreType.DMA,
            pltpu.SemaphoreType.DMA,
        ],
    )(pos, tails)


def kernel(z, edge_index, pos_edge_index):
    del z
    return _degree_gather(pos_edge_index.astype(jnp.int32),
                          edge_index.astype(jnp.int32))


# speedup vs baseline: 1.0118x; 1.0028x over previous
import jax
import jax.numpy as jnp
from jax import lax
from jax.experimental import pallas as pl
from jax.experimental.pallas import tpu as pltpu
from jax.experimental.pallas import tpu_sc as plsc

N_NODES = 10000
N_EDGES = 320000
L = 16
NC = 2
NS = 16
NPAD = 10240
SLICE = NPAD // NS
E_HIST = N_EDGES // NS
E_OUT = N_EDGES // (NC * NS)
TILE1 = 128
E_HIST_BUF = 20096
E_OUT_BUF = 10112


def _sc_body(pos_hbm, tail_hbm, out_hbm, hist_v, idx_v, tails_v, staged_v,
             red_v, outbuf_v, rows_sh, deg_sh, sem_tails, sem_pos, sem_out):
    cid = lax.axis_index("c")
    sid = lax.axis_index("s")
    wid = sid * NC + cid
    gbase = wid * E_OUT

    ones = jnp.ones((L,), jnp.float32)
    zeros = jnp.zeros((L,), jnp.float32)

    start_t = jnp.minimum((gbase // TILE1) * TILE1, N_EDGES - E_OUT_BUF)
    start_t = pl.multiple_of(start_t, TILE1)
    off_t = gbase - start_t

    start_h = jnp.minimum((sid * E_HIST // TILE1) * TILE1, N_EDGES - E_HIST_BUF)
    start_h = pl.multiple_of(start_h, TILE1)
    off_h = sid * E_HIST - start_h
    HSEGS = (80, 320, 320, 320, 210)
    HBOUND = (0, 1408, 6528, 11648, 16768, E_HIST_BUF)
    pos_cps = [
        pltpu.async_copy(
            pos_hbm.at[:, pl.ds(start_h + HBOUND[c], HBOUND[c + 1] - HBOUND[c])],
            idx_v.at[:, pl.ds(HBOUND[c], HBOUND[c + 1] - HBOUND[c])],
            sem_pos)
        for c in range(len(HSEGS))
    ]

    @plsc.parallel_loop(0, NPAD // L, unroll=4)
    def _zero(i):
        hist_v[pl.ds(i * L, L)] = zeros

    tails_cp = None
    base = 0
    for c in range(len(HSEGS)):
        pos_cps[c].wait()
        seg_base = base

        @plsc.parallel_loop(0, HSEGS[c], unroll=5)
        def _hist(e, _b=seg_base):
            plsc.addupdate_scatter(
                hist_v, [idx_v[1, pl.ds(off_h + (_b + e) * L, L)]], ones)
        base += HSEGS[c]
        if c == 1:
            tails_cp = pltpu.async_copy(
                tail_hbm.at[:, pl.ds(start_t, E_OUT_BUF)], tails_v, sem_tails)

    pltpu.sync_copy(hist_v, rows_sh.at[sid])
    plsc.subcore_barrier()

    col = sid * SLICE
    pltpu.sync_copy(rows_sh.at[:, pl.ds(col, SLICE)], staged_v)

    @plsc.parallel_loop(0, SLICE // L, unroll=2)
    def _reduce(j):
        acc = staged_v[0, pl.ds(j * L, L)]
        for r in range(1, NS):
            acc = acc + staged_v[r, pl.ds(j * L, L)]
        red_v[pl.ds(j * L, L)] = acc

    pltpu.sync_copy(red_v, deg_sh.at[pl.ds(col, SLICE)])
    plsc.subcore_barrier()

    pltpu.sync_copy(deg_sh, hist_v)
    tails_cp.wait()
    GSEGS = (320, 305)
    out_cps = []
    gb = 0
    for g in GSEGS:
        seg_base = gb

        @plsc.parallel_loop(0, g, unroll=5 if g == 305 else 4)
        def _gather(e, _b=seg_base):
            outbuf_v[pl.ds((_b + e) * L, L)] = plsc.load_gather(
                hist_v, [tails_v[1, pl.ds(off_t + (_b + e) * L, L)]])

        out_cps.append(pltpu.async_copy(
            outbuf_v.at[pl.ds(seg_base * L, g * L)],
            out_hbm.at[pl.ds(gbase + seg_base * L, g * L)], sem_out))
        gb += g
    for cp in out_cps:
        cp.wait()


@jax.jit
def _degree_gather(pos, tails):
    mesh = plsc.VectorSubcoreMesh(core_axis_name="c", subcore_axis_name="s")
    return pl.kernel(
        _sc_body,
        mesh=mesh,
        compiler_params=pltpu.CompilerParams(needs_layout_passes=False),
        out_type=jax.ShapeDtypeStruct((N_EDGES,), jnp.float32),
        scratch_types=[
            pltpu.VMEM((NPAD,), jnp.float32),
            pltpu.VMEM((2, E_HIST_BUF), jnp.int32),
            pltpu.VMEM((2, E_OUT_BUF), jnp.int32),
            pltpu.VMEM((NS, SLICE), jnp.float32),
            pltpu.VMEM((SLICE,), jnp.float32),
            pltpu.VMEM((E_OUT,), jnp.float32),
            pltpu.VMEM_SHARED((NS, NPAD), jnp.float32),
            pltpu.VMEM_SHARED((NPAD,), jnp.float32),
            pltpu.SemaphoreType.DMA,
            pltpu.SemaphoreType.DMA,
            pltpu.SemaphoreType.DMA,
        ],
    )(pos, tails)


def kernel(z, edge_index, pos_edge_index):
    del z
    return _degree_gather(pos_edge_index.astype(jnp.int32),
                          edge_index.astype(jnp.int32))
